# baseline (speedup 1.0000x reference)
import functools

import jax
import jax.numpy as jnp
from jax import lax
from jax.experimental import pallas as pl
from jax.experimental.pallas import tpu as pltpu
from jax.experimental.pallas import tpu_sc as plsc

TOP_K = 8
NUM_EXPERTS = 64
HIDDEN = 4096
TOKEN_BLOCK = 1024

NC = 2
NS = 16
LANES = 16
NW = NC * NS
TOKENS = 4 * 4096
TPW = TOKENS // NW
GROUPS = TPW // LANES


def _logits_block(x_ref, wt_ref, b_ref, out_ref):
    out_ref[...] = jnp.dot(
        x_ref[...], wt_ref[...], preferred_element_type=jnp.float32
    ) + b_ref[...]


def _tc_logits(x, wt, b2):
    T = x.shape[0]
    grid = (T // TOKEN_BLOCK,)
    return pl.pallas_call(
        _logits_block,
        grid=grid,
        in_specs=[
            pl.BlockSpec((TOKEN_BLOCK, HIDDEN), lambda i: (i, 0)),
            pl.BlockSpec((HIDDEN, NUM_EXPERTS), lambda i: (0, 0)),
            pl.BlockSpec((1, NUM_EXPERTS), lambda i: (0, 0)),
        ],
        out_specs=pl.BlockSpec((TOKEN_BLOCK, NUM_EXPERTS), lambda i: (i, 0)),
        out_shape=jax.ShapeDtypeStruct((T, NUM_EXPERTS), jnp.float32),
        compiler_params=pltpu.CompilerParams(
            dimension_semantics=("arbitrary",),
        ),
    )(x, wt, b2)


def _sc_route(logits_hbm, scores_hbm, idx_hbm, logits_v, scores_v, idx_v):
    wid = lax.axis_index("s") * NC + lax.axis_index("c")
    base = wid * TPW
    pltpu.sync_copy(
        logits_hbm.at[pl.ds(base * NUM_EXPERTS, TPW * NUM_EXPERTS)], logits_v)

    zeros16 = jnp.zeros((LANES,), jnp.float32)

    def zero_chunk(i, _):
        scores_v[pl.ds(i * LANES, LANES)] = zeros16
        return 0

    lax.fori_loop(0, TPW * NUM_EXPERTS // LANES, zero_chunk, 0)

    lane = lax.iota(jnp.int32, LANES)
    neg_inf = jnp.full((LANES,), -jnp.inf, jnp.float32)
    izero = jnp.zeros((LANES,), jnp.int32)

    def group_body(g, _):
        tok = g * LANES + lane
        row_s = tok * NUM_EXPERTS
        row_i = tok * TOP_K

        def expert_body(e, carry):
            rv = list(carry[0])
            ri = list(carry[1])
            v = plsc.load_gather(logits_v, [row_s + e])
            iv = jnp.full((LANES,), e)
            for j in range(TOP_K):
                gt = v > rv[j]
                new_rv = jnp.where(gt, v, rv[j])
                new_v = jnp.where(gt, rv[j], v)
                new_ri = jnp.where(gt, iv, ri[j])
                new_iv = jnp.where(gt, ri[j], iv)
                rv[j], v = new_rv, new_v
                ri[j], iv = new_ri, new_iv
            return (tuple(rv), tuple(ri))

        init = ((neg_inf,) * TOP_K, (izero,) * TOP_K)
        rv, ri = lax.fori_loop(0, NUM_EXPERTS, expert_body, init)

        top = rv[0]
        ev = [jnp.exp(rv[j] - top) for j in range(TOP_K)]
        denom = ev[0]
        for j in range(1, TOP_K):
            denom = denom + ev[j]
        inv = 1.0 / denom
        for j in range(TOP_K):
            plsc.store_scatter(scores_v, [row_s + ri[j]], ev[j] * inv)
            plsc.store_scatter(idx_v, [row_i + j], ri[j])
        return 0

    lax.fori_loop(0, GROUPS, group_body, 0)

    pltpu.sync_copy(
        scores_v, scores_hbm.at[pl.ds(base * NUM_EXPERTS, TPW * NUM_EXPERTS)])
    pltpu.sync_copy(idx_v, idx_hbm.at[pl.ds(base * TOP_K, TPW * TOP_K)])


_sc_route_call = pl.kernel(
    _sc_route,
    out_type=(
        jax.ShapeDtypeStruct((TOKENS * NUM_EXPERTS,), jnp.float32),
        jax.ShapeDtypeStruct((TOKENS * TOP_K,), jnp.int32),
    ),
    mesh=plsc.VectorSubcoreMesh(
        core_axis_name="c", subcore_axis_name="s"),
    compiler_params=pltpu.CompilerParams(needs_layout_passes=False),
    scratch_types=(
        pltpu.VMEM((TPW * NUM_EXPERTS,), jnp.float32),
        pltpu.VMEM((TPW * NUM_EXPERTS,), jnp.float32),
        pltpu.VMEM((TPW * TOP_K,), jnp.int32),
    ),
)


@functools.partial(jax.jit, static_argnames=())
def kernel(hidden_states, W, b):
    B, S, H = hidden_states.shape
    T = B * S
    x = hidden_states.reshape(T, H)
    wt = W.T
    b2 = b.reshape(1, NUM_EXPERTS)
    logits = _tc_logits(x, wt, b2)
    scores_flat, idx_flat = _sc_route_call(logits.reshape(-1))
    return (scores_flat.reshape(T, NUM_EXPERTS),
            idx_flat.reshape(T, TOP_K))


# speedup vs baseline: 1.1224x; 1.1224x over previous
import functools

import jax
import jax.numpy as jnp
from jax.experimental import pallas as pl
from jax.experimental.pallas import tpu as pltpu

TOP_K = 8
NUM_EXPERTS = 64
HIDDEN = 4096
TOKEN_BLOCK = 1024
H_CHUNK = 1024
N_H = HIDDEN // H_CHUNK


def _router_block(x_ref, wt_ref, b_ref, scores_ref, idx_ref, acc_ref):
    j = pl.program_id(1)
    part = jnp.dot(x_ref[...], wt_ref[...], preferred_element_type=jnp.float32)

    @pl.when(j == 0)
    def _init():
        acc_ref[...] = part

    @pl.when(j != 0)
    def _accum():
        acc_ref[...] = acc_ref[...] + part

    @pl.when(j == N_H - 1)
    def _epilogue():
        logits = acc_ref[...] + b_ref[...]

        tb = logits.shape[0]
        e_iota = jax.lax.broadcasted_iota(
            jnp.int32, (tb, NUM_EXPERTS), 1).astype(jnp.float32)

        vals = logits
        top_max = None
        idx_cols = []
        for k in range(TOP_K):
            m = jnp.max(vals, axis=1, keepdims=True)
            hit = vals == m
            idx = jnp.min(jnp.where(hit, e_iota, float(NUM_EXPERTS)), axis=1,
                          keepdims=True)
            vals = jnp.where(e_iota == idx, -jnp.inf, vals)
            if k == 0:
                top_max = m
            idx_cols.append(idx)

        selected = vals == -jnp.inf
        unnorm = jnp.where(selected, jnp.exp(logits - top_max), 0.0)
        denom = jnp.sum(unnorm, axis=1, keepdims=True)
        scores_ref[...] = unnorm / denom
        idx_ref[...] = jnp.concatenate(idx_cols, axis=1).astype(jnp.int32)


@functools.partial(jax.jit, static_argnames=())
def kernel(hidden_states, W, b):
    B, S, H = hidden_states.shape
    T = B * S
    x = hidden_states.reshape(T, H)
    wt = W.T
    b2 = b.reshape(1, NUM_EXPERTS)

    grid = (T // TOKEN_BLOCK, N_H)
    scores, indices = pl.pallas_call(
        _router_block,
        grid=grid,
        in_specs=[
            pl.BlockSpec((TOKEN_BLOCK, H_CHUNK), lambda i, j: (i, j)),
            pl.BlockSpec((H_CHUNK, NUM_EXPERTS), lambda i, j: (j, 0)),
            pl.BlockSpec((1, NUM_EXPERTS), lambda i, j: (0, 0)),
        ],
        out_specs=[
            pl.BlockSpec((TOKEN_BLOCK, NUM_EXPERTS), lambda i, j: (i, 0)),
            pl.BlockSpec((TOKEN_BLOCK, TOP_K), lambda i, j: (i, 0)),
        ],
        out_shape=[
            jax.ShapeDtypeStruct((T, NUM_EXPERTS), jnp.float32),
            jax.ShapeDtypeStruct((T, TOP_K), jnp.int32),
        ],
        scratch_shapes=[pltpu.VMEM((TOKEN_BLOCK, NUM_EXPERTS), jnp.float32)],
        compiler_params=pltpu.CompilerParams(
            dimension_semantics=("arbitrary", "arbitrary"),
        ),
    )(x, wt, b2)
    return scores, indices


# speedup vs baseline: 1.6733x; 1.4908x over previous
import functools

import jax
import jax.numpy as jnp
from jax.experimental import pallas as pl
from jax.experimental.pallas import tpu as pltpu

TOP_K = 8
NUM_EXPERTS = 64
HIDDEN = 4096
TOKEN_BLOCK = 1024


def _router_block(x_ref, wt_ref, b_ref, scores_ref, idx_ref):
    x = x_ref[...]
    logits = jnp.dot(x, wt_ref[...], preferred_element_type=jnp.float32)
    logits = logits + b_ref[...]

    tb = logits.shape[0]
    e_iota = jax.lax.broadcasted_iota(
        jnp.int32, (tb, NUM_EXPERTS), 1).astype(jnp.float32)

    vals = logits
    top_max = None
    idx_cols = []
    for k in range(TOP_K):
        m = jnp.max(vals, axis=1, keepdims=True)
        hit = vals == m
        idx = jnp.min(jnp.where(hit, e_iota, float(NUM_EXPERTS)), axis=1,
                      keepdims=True)
        vals = jnp.where(e_iota == idx, -jnp.inf, vals)
        if k == 0:
            top_max = m
        idx_cols.append(idx)

    selected = vals == -jnp.inf
    unnorm = jnp.where(selected, jnp.exp(logits - top_max), 0.0)
    denom = jnp.sum(unnorm, axis=1, keepdims=True)
    scores_ref[...] = unnorm / denom
    idx_ref[...] = jnp.concatenate(idx_cols, axis=1).astype(jnp.int32)


@functools.partial(jax.jit, static_argnames=())
def kernel(hidden_states, W, b):
    B, S, H = hidden_states.shape
    T = B * S
    x = hidden_states.reshape(T, H)
    wt = W.T
    b2 = b.reshape(1, NUM_EXPERTS)

    grid = (T // TOKEN_BLOCK,)
    scores, indices = pl.pallas_call(
        _router_block,
        grid=grid,
        in_specs=[
            pl.BlockSpec((TOKEN_BLOCK, H), lambda i: (i, 0)),
            pl.BlockSpec((H, NUM_EXPERTS), lambda i: (0, 0)),
            pl.BlockSpec((1, NUM_EXPERTS), lambda i: (0, 0)),
        ],
        out_specs=[
            pl.BlockSpec((TOKEN_BLOCK, NUM_EXPERTS), lambda i: (i, 0)),
            pl.BlockSpec((TOKEN_BLOCK, TOP_K), lambda i: (i, 0)),
        ],
        out_shape=[
            jax.ShapeDtypeStruct((T, NUM_EXPERTS), jnp.float32),
            jax.ShapeDtypeStruct((T, TOP_K), jnp.int32),
        ],
        compiler_params=pltpu.CompilerParams(
            dimension_semantics=("arbitrary",),
        ),
    )(x, wt, b2)
    return scores, indices
